# baseline (speedup 1.0000x reference)
import functools

import jax
import jax.numpy as jnp
from jax import lax
from jax.experimental import pallas as pl
from jax.experimental.pallas import tpu as pltpu
from jax.experimental.pallas import tpu_sc as plsc

BATCH = 4096
SEQ = 200
DIM = 64
NLANE = 16
NC = 2
NS = 16
NW = NC * NS
BPW = BATCH // NW
DT = DIM // 8
NBUF = 8
NST = 2


def _sc_embed(x_t, class_table, pos_table):
    mesh = plsc.VectorSubcoreMesh(core_axis_name="c", subcore_axis_name="s")

    @functools.partial(
        pl.kernel,
        out_type=jax.ShapeDtypeStruct((SEQ, DT, NW, 8, 128), jnp.float32),
        mesh=mesh,
        compiler_params=pltpu.CompilerParams(
            use_tc_tiling_on_sc=False, needs_layout_passes=False),
        scratch_types=[
            pltpu.VMEM((SEQ, BPW), jnp.int32),
            pltpu.VMEM((SEQ, DIM), jnp.float32),
            [pltpu.VMEM((BPW, DIM), jnp.float32) for _ in range(NBUF)],
            [pltpu.VMEM((DIM, 129), jnp.float32) for _ in range(NST)],
            [pltpu.SemaphoreType.DMA for _ in range(NBUF)],
            [pltpu.SemaphoreType.DMA for _ in range(NST)],
        ],
    )
    def kern(xt_hbm, tab_hbm, pos_hbm, out_hbm, idx_v, pos_v, bufs, stg,
             gsems, osems):
        wid = lax.axis_index("s") * NC + lax.axis_index("c")
        bcol = pl.multiple_of(wid * BPW, 8)
        pltpu.sync_copy(xt_hbm.at[:, pl.ds(bcol, BPW)], idx_v)
        pltpu.sync_copy(pos_hbm, pos_v)

        lane = lax.broadcasted_iota(jnp.int32, (NLANE,), 0)

        def start_gather(s, b):
            pltpu.async_copy(tab_hbm.at[idx_v.at[s]], bufs[b], gsems[b])

        def wait_gather(b):
            pltpu.make_async_copy(
                tab_hbm.at[pl.ds(0, BPW)], bufs[b], gsems[b]).wait()

        def wait_tiles(sb):
            for ti in range(DT):
                pltpu.make_async_copy(
                    stg[sb].at[pl.ds(ti * 8, 8), pl.ds(0, 128)],
                    out_hbm.at[0, ti, 0], osems[sb]).wait()

        for s0 in range(NBUF - 1):
            start_gather(s0, s0)

        def chunk_step(s, b, sb):
            wait_gather(b)

            @pl.when(s + (NBUF - 1) < SEQ)
            def _():
                start_gather(s + (NBUF - 1), (b + NBUF - 1) % NBUF)

            @pl.when(s >= NST)
            def _():
                wait_tiles(sb)

            pos_r = [pos_v[s, pl.ds(j * NLANE, NLANE)]
                     for j in range(DIM // NLANE)]
            rowidx = [lane + j * NLANE for j in range(DIM // NLANE)]

            @plsc.parallel_loop(0, BPW, unroll=4)
            def _(r):
                col = jnp.broadcast_to(r, (NLANE,))
                for j in range(DIM // NLANE):
                    v = bufs[b][r, pl.ds(j * NLANE, NLANE)] + pos_r[j]
                    plsc.store_scatter(stg[sb], [rowidx[j], col], v)

            for ti in range(DT):
                pltpu.async_copy(
                    stg[sb].at[pl.ds(ti * 8, 8), pl.ds(0, 128)],
                    out_hbm.at[s, ti, wid], osems[sb])

        @pl.loop(0, SEQ // NBUF)
        def _(g):
            for bb in range(NBUF):
                chunk_step(g * NBUF + bb, bb, bb % NST)

        for sb in range(NST):
            wait_tiles(sb)

    return kern(x_t, class_table, pos_table)


def kernel(x, class_table, pos_table):
    x_t = jnp.swapaxes(x, 0, 1).astype(jnp.int32)
    y5 = _sc_embed(x_t, class_table, pos_table)
    return y5.transpose(2, 4, 0, 1, 3).reshape(BATCH, SEQ, DIM)


# speedup vs baseline: 1.0067x; 1.0067x over previous
import functools

import jax
import jax.numpy as jnp
from jax import lax
from jax.experimental import pallas as pl
from jax.experimental.pallas import tpu as pltpu
from jax.experimental.pallas import tpu_sc as plsc

BATCH = 4096
SEQ = 200
DIM = 64
NLANE = 16
NC = 2
NS = 16
NW = NC * NS
BPW = BATCH // NW
DT = DIM // 8
NBUF = 4
NST = 2


def _sc_embed(x_t, class_table, pos_table):
    mesh = plsc.VectorSubcoreMesh(core_axis_name="c", subcore_axis_name="s")

    @functools.partial(
        pl.kernel,
        out_type=jax.ShapeDtypeStruct((SEQ, DT, NW, 8, 128), jnp.float32),
        mesh=mesh,
        compiler_params=pltpu.CompilerParams(
            use_tc_tiling_on_sc=False, needs_layout_passes=False),
        scratch_types=[
            pltpu.VMEM((SEQ, BPW), jnp.int32),
            pltpu.VMEM((SEQ, DIM), jnp.float32),
            [pltpu.VMEM((BPW, DIM), jnp.float32) for _ in range(NBUF)],
            [pltpu.VMEM((DIM, 129), jnp.float32) for _ in range(NST)],
            [pltpu.SemaphoreType.DMA for _ in range(NBUF)],
            [pltpu.SemaphoreType.DMA for _ in range(NST)],
        ],
    )
    def kern(xt_hbm, tab_hbm, pos_hbm, out_hbm, idx_v, pos_v, bufs, stg,
             gsems, osems):
        wid = lax.axis_index("s") * NC + lax.axis_index("c")
        bcol = pl.multiple_of(wid * BPW, 8)
        pltpu.sync_copy(xt_hbm.at[:, pl.ds(bcol, BPW)], idx_v)
        pltpu.sync_copy(pos_hbm, pos_v)

        lane = lax.broadcasted_iota(jnp.int32, (NLANE,), 0)

        def start_gather(s, b):
            pltpu.async_copy(tab_hbm.at[idx_v.at[s]], bufs[b], gsems[b])

        def wait_gather(b):
            pltpu.make_async_copy(
                tab_hbm.at[pl.ds(0, BPW)], bufs[b], gsems[b]).wait()

        def wait_tiles(sb):
            for ti in range(DT):
                pltpu.make_async_copy(
                    stg[sb].at[pl.ds(ti * 8, 8), pl.ds(0, 128)],
                    out_hbm.at[0, ti, 0], osems[sb]).wait()

        for s0 in range(NBUF - 1):
            start_gather(s0, s0)

        def chunk_step(s, b, sb):
            wait_gather(b)

            @pl.when(s + (NBUF - 1) < SEQ)
            def _():
                start_gather(s + (NBUF - 1), (b + NBUF - 1) % NBUF)

            @pl.when(s >= NST)
            def _():
                wait_tiles(sb)

            pos_r = [pos_v[s, pl.ds(j * NLANE, NLANE)]
                     for j in range(DIM // NLANE)]
            rowidx = [lane + j * NLANE for j in range(DIM // NLANE)]

            @plsc.parallel_loop(0, BPW, unroll=4)
            def _(r):
                col = jnp.broadcast_to(r, (NLANE,))
                for j in range(DIM // NLANE):
                    v = bufs[b][r, pl.ds(j * NLANE, NLANE)] + pos_r[j]
                    plsc.store_scatter(stg[sb], [rowidx[j], col], v)

            for ti in range(DT):
                pltpu.async_copy(
                    stg[sb].at[pl.ds(ti * 8, 8), pl.ds(0, 128)],
                    out_hbm.at[s, ti, wid], osems[sb])

        @pl.loop(0, SEQ // NBUF)
        def _(g):
            for bb in range(NBUF):
                chunk_step(g * NBUF + bb, bb, bb % NST)

        for sb in range(NST):
            wait_tiles(sb)

    return kern(x_t, class_table, pos_table)


def kernel(x, class_table, pos_table):
    x_t = jnp.swapaxes(x, 0, 1).astype(jnp.int32)
    y5 = _sc_embed(x_t, class_table, pos_table)
    return y5.transpose(2, 4, 0, 1, 3).reshape(BATCH, SEQ, DIM)
